# baseline (speedup 1.0000x reference)
import functools

import jax
import jax.numpy as jnp
from jax.experimental import pallas as pl
from jax.experimental.pallas import tpu as pltpu
from jax.experimental.pallas import tpu_sc as plsc

SD = 128
HL = 2
N_NODES = 10000
N_EDGES = 320000
N_GRAPHS = 64
VOCAB = 1340
N_CLASSES = 41

NC = 2
NS = 16
NW = NC * NS
EBLK = 128
TBLKS = 2560
B0 = 80
B1 = 80
CHB = 16
NCH = 80 // CHB
BMAX = max(B0, B1)
E_PAD = TBLKS * EBLK
DUMMY = N_NODES
ROWS_PER_TILE = 632
ACC_ROWS = NS * ROWS_PER_TILE
XB = 3
X_PAD = NW * XB * EBLK

NB = 400
NBLK = N_NODES // NB

_vec_mesh = plsc.VectorSubcoreMesh(core_axis_name="core",
                                   subcore_axis_name="subcore")


def _embed(emb, xi):
  @functools.partial(
      pl.kernel,
      out_type=jax.ShapeDtypeStruct((X_PAD, SD), jnp.float32),
      mesh=_vec_mesh,
      scratch_types=[
          pltpu.VMEM((XB, EBLK), jnp.int32),
          pltpu.VMEM((EBLK, SD), jnp.float32),
      ],
  )
  def embed_kernel(emb_hbm, xi_hbm, out_hbm, xi_v, rows_v):
    c = jax.lax.axis_index("core")
    s = jax.lax.axis_index("subcore")
    wid = c * NS + s
    pltpu.sync_copy(xi_hbm.at[wid], xi_v)

    @pl.loop(0, XB)
    def _(j):
      pltpu.sync_copy(emb_hbm.at[xi_v.at[j]], rows_v)
      pltpu.sync_copy(rows_v, out_hbm.at[pl.ds(wid * XB * EBLK + j * EBLK,
                                               EBLK)])

  return embed_kernel(emb, xi)


def _agg(h, srcp, dstp, zero_rows):
  @functools.partial(
      pl.kernel,
      out_type=jax.ShapeDtypeStruct((NC, ACC_ROWS, SD), jnp.float32),
      mesh=_vec_mesh,
      scratch_types=[
          pltpu.VMEM_SHARED((ACC_ROWS, SD), jnp.float32),
          pltpu.VMEM((CHB, EBLK), jnp.int32),
          pltpu.VMEM((CHB, EBLK), jnp.int32),
          pltpu.VMEM((2, EBLK, SD), jnp.float32),
      ],
  )
  def agg_kernel(h_hbm, src_hbm, dst_hbm, zero_hbm, agg_hbm,
                 acc, sidx, didx, rows_d):
    c = jax.lax.axis_index("core")
    s = jax.lax.axis_index("subcore")
    wid = c * NS + s
    pltpu.sync_copy(zero_hbm, acc.at[pl.ds(s * ROWS_PER_TILE, ROWS_PER_TILE)])
    plsc.subcore_barrier()

    @pl.loop(0, NCH)
    def _(ck):
      pltpu.sync_copy(src_hbm.at[wid, pl.ds(ck * CHB, CHB)], sidx)
      pltpu.sync_copy(dst_hbm.at[wid, pl.ds(ck * CHB, CHB)], didx)

      @plsc.parallel_loop(0, CHB, unroll=2)
      def _(b):
        p = jax.lax.rem(b, 2)
        pltpu.sync_copy(h_hbm.at[sidx.at[b]], rows_d.at[p])
        pltpu.sync_copy(rows_d.at[p], acc.at[didx.at[b]], add=True)

    plsc.subcore_barrier()
    pltpu.sync_copy(acc.at[pl.ds(s * ROWS_PER_TILE, ROWS_PER_TILE)],
                    agg_hbm.at[c, pl.ds(s * ROWS_PER_TILE, ROWS_PER_TILE)])

  return agg_kernel(h, srcp, dstp, zero_rows)


def _mlp_body(h_ref, agg_ref, w1_ref, b1_ref, w2_ref, b2_ref,
              out_ref, st_ref):
  i = pl.program_id(0)
  z = h_ref[...] + agg_ref[0] + agg_ref[1]
  u = jnp.maximum(
      jnp.dot(z, w1_ref[...], preferred_element_type=jnp.float32)
      + b1_ref[...], 0.0)
  v = (jnp.dot(u, w2_ref[...], preferred_element_type=jnp.float32)
       + b2_ref[...])
  hout = jnp.maximum(v, 0.0)
  out_ref[...] = hout
  su = jnp.sum(hout, axis=0, keepdims=True)
  sq = jnp.sum(hout * hout, axis=0, keepdims=True)
  upd = jnp.concatenate([su, sq, jnp.zeros((6, SD), jnp.float32)], axis=0)

  @pl.when(i == 0)
  def _():
    st_ref[...] = upd

  @pl.when(i > 0)
  def _():
    st_ref[...] = st_ref[...] + upd


def _mlp(h, agg, w1, b1, w2, b2):
  return pl.pallas_call(
      _mlp_body,
      grid=(NBLK,),
      in_specs=[
          pl.BlockSpec((NB, SD), lambda i: (i, 0)),
          pl.BlockSpec((NC, NB, SD), lambda i: (0, i, 0)),
          pl.BlockSpec((SD, SD), lambda i: (0, 0)),
          pl.BlockSpec((1, SD), lambda i: (0, 0)),
          pl.BlockSpec((SD, SD), lambda i: (0, 0)),
          pl.BlockSpec((1, SD), lambda i: (0, 0)),
      ],
      out_specs=[
          pl.BlockSpec((NB, SD), lambda i: (i, 0)),
          pl.BlockSpec((8, SD), lambda i: (0, 0)),
      ],
      out_shape=[
          jax.ShapeDtypeStruct((N_NODES, SD), jnp.float32),
          jax.ShapeDtypeStruct((8, SD), jnp.float32),
      ],
  )(h, agg, w1, b1, w2, b2)


def _norm_body(v_ref, st_ref, g_ref, b_ref, out_ref):
  inv_n = 1.0 / N_NODES
  mean = st_ref[0:1, :] * inv_n
  ex2 = st_ref[1:2, :] * inv_n
  var = ex2 - mean * mean
  a = g_ref[...] * jax.lax.rsqrt(var + 1e-5)
  b = b_ref[...] - mean * a
  out_ref[...] = v_ref[...] * a + b


def _norm(v, st, gamma, beta):
  return pl.pallas_call(
      _norm_body,
      grid=(NBLK,),
      in_specs=[
          pl.BlockSpec((NB, SD), lambda i: (i, 0)),
          pl.BlockSpec((8, SD), lambda i: (0, 0)),
          pl.BlockSpec((1, SD), lambda i: (0, 0)),
          pl.BlockSpec((1, SD), lambda i: (0, 0)),
      ],
      out_specs=pl.BlockSpec((NB, SD), lambda i: (i, 0)),
      out_shape=jax.ShapeDtypeStruct((N_NODES, SD), jnp.float32),
  )(v, st, gamma, beta)


def _final_body(h_ref, agg_ref, w1_ref, b1_ref, w2_ref, b2_ref, batch_ref,
                fc1w_ref, fc1b_ref, fc2w_ref, fc2b_ref, out_ref, acc_ref):
  i = pl.program_id(0)
  z = h_ref[...] + agg_ref[0] + agg_ref[1]
  u = jnp.maximum(
      jnp.dot(z, w1_ref[...], preferred_element_type=jnp.float32)
      + b1_ref[...], 0.0)
  v = (jnp.dot(u, w2_ref[...], preferred_element_type=jnp.float32)
       + b2_ref[...])
  hout = jnp.maximum(v, 0.0)
  bids = batch_ref[0, 0, :]
  rows = jax.lax.broadcasted_iota(jnp.int32, (N_GRAPHS, NB), 0)
  onehot = (rows == bids[None, :]).astype(jnp.float32)
  contrib = jnp.dot(onehot, hout, preferred_element_type=jnp.float32)

  @pl.when(i == 0)
  def _():
    acc_ref[...] = contrib

  @pl.when(i > 0)
  def _():
    acc_ref[...] = acc_ref[...] + contrib

  @pl.when(i == NBLK - 1)
  def _():
    g = jnp.maximum(
        jnp.dot(acc_ref[...], fc1w_ref[...],
                preferred_element_type=jnp.float32) + fc1b_ref[...], 0.0)
    out_ref[...] = (jnp.dot(g, fc2w_ref[...],
                            preferred_element_type=jnp.float32)
                    + fc2b_ref[...])


def _final(h, agg, w1, b1, w2, b2, batch3, fc1w, fc1b, fc2w, fc2b):
  return pl.pallas_call(
      _final_body,
      grid=(NBLK,),
      in_specs=[
          pl.BlockSpec((NB, SD), lambda i: (i, 0)),
          pl.BlockSpec((NC, NB, SD), lambda i: (0, i, 0)),
          pl.BlockSpec((SD, SD), lambda i: (0, 0)),
          pl.BlockSpec((1, SD), lambda i: (0, 0)),
          pl.BlockSpec((SD, SD), lambda i: (0, 0)),
          pl.BlockSpec((1, SD), lambda i: (0, 0)),
          pl.BlockSpec((1, 1, NB), lambda i: (i, 0, 0)),
          pl.BlockSpec((SD, SD), lambda i: (0, 0)),
          pl.BlockSpec((1, SD), lambda i: (0, 0)),
          pl.BlockSpec((SD, SD), lambda i: (0, 0)),
          pl.BlockSpec((1, SD), lambda i: (0, 0)),
      ],
      out_specs=pl.BlockSpec((N_GRAPHS, SD), lambda i: (0, 0)),
      out_shape=jax.ShapeDtypeStruct((N_GRAPHS, SD), jnp.float32),
      scratch_shapes=[pltpu.VMEM((N_GRAPHS, SD), jnp.float32)],
  )(h, agg, w1, b1, w2, b2, batch3, fc1w, fc1b, fc2w, fc2b)


def kernel(x, edge_index, batch, emb, nn_in_W1, nn_in_b1, nn_in_W2, nn_in_b2,
           bn_gamma, bn_beta, nn_out_W1, nn_out_b1, nn_out_W2, nn_out_b2,
           fc1_W, fc1_b, fc2_W, fc2_b):
  f32 = jnp.float32
  src = edge_index[0]
  dst = edge_index[1]

  def edge_layout(idx, fill):
    flat = jnp.concatenate(
        [idx, jnp.full((E_PAD - N_EDGES,), fill, jnp.int32)]
    ).reshape(TBLKS, EBLK)
    c0 = flat[:NS * B0].reshape(NS, B0, EBLK)
    c1 = flat[NS * B0:].reshape(NS, B1, EBLK)
    pad0 = jnp.full((NS, BMAX - B0, EBLK), fill, jnp.int32)
    pad1 = jnp.full((NS, BMAX - B1, EBLK), fill, jnp.int32)
    return jnp.concatenate(
        [jnp.concatenate([c0, pad0], axis=1),
         jnp.concatenate([c1, pad1], axis=1)], axis=0)

  srcp = edge_layout(src, 0)
  dstp = edge_layout(dst, DUMMY)
  xi = jnp.concatenate(
      [jnp.squeeze(x, -1), jnp.zeros((X_PAD - N_NODES,), jnp.int32)]
  ).reshape(NW, XB, EBLK)
  zero_rows = jnp.zeros((ROWS_PER_TILE, SD), f32)

  w1i = nn_in_W1.T
  w2i = nn_in_W2.T
  w1o = nn_out_W1.T
  w2o = nn_out_W2.T
  b1i = nn_in_b1.reshape(1, SD)
  b2i = nn_in_b2.reshape(1, SD)
  b1o = nn_out_b1.reshape(1, SD)
  b2o = nn_out_b2.reshape(1, SD)
  gam = bn_gamma.reshape(1, SD)
  bet = bn_beta.reshape(1, SD)
  fc1T = fc1_W.T
  fc1b2 = fc1_b.reshape(1, SD)
  fc2T = jnp.zeros((SD, SD), f32).at[:, :N_CLASSES].set(fc2_W.T)
  fc2b2 = jnp.zeros((1, SD), f32).at[0, :N_CLASSES].set(fc2_b)
  batch3 = batch.reshape(NBLK, 1, NB)

  h = _embed(emb, xi)
  for _ in range(1 + HL):
    agg = _agg(h, srcp, dstp, zero_rows)
    v, st = _mlp(h, agg, w1i, b1i, w2i, b2i)
    h = _norm(v, st, gam, bet)
  agg = _agg(h, srcp, dstp, zero_rows)
  out = _final(h, agg, w1o, b1o, w2o, b2o, batch3, fc1T, fc1b2, fc2T, fc2b2)
  return out[:, :N_CLASSES]


# speedup vs baseline: 1.1401x; 1.1401x over previous
import functools

import jax
import jax.numpy as jnp
from jax.experimental import pallas as pl
from jax.experimental.pallas import tpu as pltpu
from jax.experimental.pallas import tpu_sc as plsc

SD = 128
HL = 2
N_NODES = 10000
N_EDGES = 320000
N_GRAPHS = 64
VOCAB = 1340
N_CLASSES = 41

NC = 2
NS = 16
NW = NC * NS
EBLK = 128
BLKS = 80
E_PAD = NW * BLKS * EBLK
DUMMY = N_NODES
ROWS_PER_TILE = 632
ACC_ROWS = NS * ROWS_PER_TILE
XB = 3
X_PAD = NW * XB * EBLK

NB = 400
NBLK = N_NODES // NB

_vec_mesh = plsc.VectorSubcoreMesh(core_axis_name="core",
                                   subcore_axis_name="subcore")


def _embed(emb, xi):
  @functools.partial(
      pl.kernel,
      out_type=jax.ShapeDtypeStruct((X_PAD, SD), jnp.float32),
      mesh=_vec_mesh,
      scratch_types=[
          pltpu.VMEM((XB, EBLK), jnp.int32),
          pltpu.VMEM((EBLK, SD), jnp.float32),
      ],
  )
  def embed_kernel(emb_hbm, xi_hbm, out_hbm, xi_v, rows_v):
    c = jax.lax.axis_index("core")
    s = jax.lax.axis_index("subcore")
    wid = c * NS + s
    pltpu.sync_copy(xi_hbm.at[wid], xi_v)

    @pl.loop(0, XB)
    def _(j):
      pltpu.sync_copy(emb_hbm.at[xi_v.at[j]], rows_v)
      pltpu.sync_copy(rows_v, out_hbm.at[pl.ds(wid * XB * EBLK + j * EBLK,
                                               EBLK)])

  return embed_kernel(emb, xi)


def _agg(h, srcp, dstp, zero_rows):
  @functools.partial(
      pl.kernel,
      out_type=jax.ShapeDtypeStruct((NC, ACC_ROWS, SD), jnp.float32),
      mesh=_vec_mesh,
      scratch_types=[
          pltpu.VMEM_SHARED((ACC_ROWS, SD), jnp.float32),
          pltpu.VMEM((BLKS, EBLK), jnp.int32),
          pltpu.VMEM((BLKS, EBLK), jnp.int32),
          pltpu.VMEM((EBLK, SD), jnp.float32),
      ],
  )
  def agg_kernel(h_hbm, src_hbm, dst_hbm, zero_hbm, agg_hbm,
                 acc, src_v, dst_v, rows0):
    c = jax.lax.axis_index("core")
    s = jax.lax.axis_index("subcore")
    wid = c * NS + s
    pltpu.sync_copy(zero_hbm, acc.at[pl.ds(s * ROWS_PER_TILE, ROWS_PER_TILE)])
    pltpu.sync_copy(src_hbm.at[wid], src_v)
    pltpu.sync_copy(dst_hbm.at[wid], dst_v)
    plsc.subcore_barrier()

    @pl.loop(0, BLKS)
    def _(j):
      pltpu.sync_copy(h_hbm.at[src_v.at[j]], rows0)
      pltpu.sync_copy(rows0, acc.at[dst_v.at[j]], add=True)

    plsc.subcore_barrier()
    pltpu.sync_copy(acc.at[pl.ds(s * ROWS_PER_TILE, ROWS_PER_TILE)],
                    agg_hbm.at[c, pl.ds(s * ROWS_PER_TILE, ROWS_PER_TILE)])

  return agg_kernel(h, srcp, dstp, zero_rows)


def _mlp_body(h_ref, agg_ref, w1_ref, b1_ref, w2_ref, b2_ref, g_ref, be_ref,
              out_ref, st_ref):
  ph = pl.program_id(0)
  i = pl.program_id(1)

  @pl.when(ph == 0)
  def _():
    z = h_ref[...] + agg_ref[0] + agg_ref[1]
    u = jnp.maximum(
        jnp.dot(z, w1_ref[...], preferred_element_type=jnp.float32)
        + b1_ref[...], 0.0)
    v = (jnp.dot(u, w2_ref[...], preferred_element_type=jnp.float32)
         + b2_ref[...])
    hout = jnp.maximum(v, 0.0)
    out_ref[...] = hout
    su = jnp.sum(hout, axis=0, keepdims=True)
    sq = jnp.sum(hout * hout, axis=0, keepdims=True)
    upd = jnp.concatenate([su, sq, jnp.zeros((6, SD), jnp.float32)], axis=0)

    @pl.when(i == 0)
    def _():
      st_ref[...] = upd

    @pl.when(i > 0)
    def _():
      st_ref[...] = st_ref[...] + upd

  @pl.when(ph == 1)
  def _():
    inv_n = 1.0 / N_NODES
    mean = st_ref[0:1, :] * inv_n
    ex2 = st_ref[1:2, :] * inv_n
    var = ex2 - mean * mean
    a = g_ref[...] * jax.lax.rsqrt(var + 1e-5)
    b = be_ref[...] - mean * a
    out_ref[...] = out_ref[...] * a + b


def _mlp_bn(h, agg, w1, b1, w2, b2, gamma, beta):
  return pl.pallas_call(
      _mlp_body,
      grid=(2, NBLK),
      in_specs=[
          pl.BlockSpec((NB, SD), lambda p, i: (i, 0)),
          pl.BlockSpec((NC, NB, SD), lambda p, i: (0, i, 0)),
          pl.BlockSpec((SD, SD), lambda p, i: (0, 0)),
          pl.BlockSpec((1, SD), lambda p, i: (0, 0)),
          pl.BlockSpec((SD, SD), lambda p, i: (0, 0)),
          pl.BlockSpec((1, SD), lambda p, i: (0, 0)),
          pl.BlockSpec((1, SD), lambda p, i: (0, 0)),
          pl.BlockSpec((1, SD), lambda p, i: (0, 0)),
      ],
      out_specs=pl.BlockSpec((NB, SD), lambda p, i: (i, 0)),
      out_shape=jax.ShapeDtypeStruct((N_NODES, SD), jnp.float32),
      scratch_shapes=[pltpu.VMEM((8, SD), jnp.float32)],
  )(h, agg, w1, b1, w2, b2, gamma, beta)


def _final_body(h_ref, agg_ref, w1_ref, b1_ref, w2_ref, b2_ref, batch_ref,
                fc1w_ref, fc1b_ref, fc2w_ref, fc2b_ref, out_ref, acc_ref):
  i = pl.program_id(0)
  z = h_ref[...] + agg_ref[0] + agg_ref[1]
  u = jnp.maximum(
      jnp.dot(z, w1_ref[...], preferred_element_type=jnp.float32)
      + b1_ref[...], 0.0)
  v = (jnp.dot(u, w2_ref[...], preferred_element_type=jnp.float32)
       + b2_ref[...])
  hout = jnp.maximum(v, 0.0)
  bids = batch_ref[0, 0, :]
  rows = jax.lax.broadcasted_iota(jnp.int32, (N_GRAPHS, NB), 0)
  onehot = (rows == bids[None, :]).astype(jnp.float32)
  contrib = jnp.dot(onehot, hout, preferred_element_type=jnp.float32)

  @pl.when(i == 0)
  def _():
    acc_ref[...] = contrib

  @pl.when(i > 0)
  def _():
    acc_ref[...] = acc_ref[...] + contrib

  @pl.when(i == NBLK - 1)
  def _():
    g = jnp.maximum(
        jnp.dot(acc_ref[...], fc1w_ref[...],
                preferred_element_type=jnp.float32) + fc1b_ref[...], 0.0)
    out_ref[...] = (jnp.dot(g, fc2w_ref[...],
                            preferred_element_type=jnp.float32)
                    + fc2b_ref[...])


def _final(h, agg, w1, b1, w2, b2, batch3, fc1w, fc1b, fc2w, fc2b):
  return pl.pallas_call(
      _final_body,
      grid=(NBLK,),
      in_specs=[
          pl.BlockSpec((NB, SD), lambda i: (i, 0)),
          pl.BlockSpec((NC, NB, SD), lambda i: (0, i, 0)),
          pl.BlockSpec((SD, SD), lambda i: (0, 0)),
          pl.BlockSpec((1, SD), lambda i: (0, 0)),
          pl.BlockSpec((SD, SD), lambda i: (0, 0)),
          pl.BlockSpec((1, SD), lambda i: (0, 0)),
          pl.BlockSpec((1, 1, NB), lambda i: (i, 0, 0)),
          pl.BlockSpec((SD, SD), lambda i: (0, 0)),
          pl.BlockSpec((1, SD), lambda i: (0, 0)),
          pl.BlockSpec((SD, SD), lambda i: (0, 0)),
          pl.BlockSpec((1, SD), lambda i: (0, 0)),
      ],
      out_specs=pl.BlockSpec((N_GRAPHS, SD), lambda i: (0, 0)),
      out_shape=jax.ShapeDtypeStruct((N_GRAPHS, SD), jnp.float32),
      scratch_shapes=[pltpu.VMEM((N_GRAPHS, SD), jnp.float32)],
  )(h, agg, w1, b1, w2, b2, batch3, fc1w, fc1b, fc2w, fc2b)


def kernel(x, edge_index, batch, emb, nn_in_W1, nn_in_b1, nn_in_W2, nn_in_b2,
           bn_gamma, bn_beta, nn_out_W1, nn_out_b1, nn_out_W2, nn_out_b2,
           fc1_W, fc1_b, fc2_W, fc2_b):
  f32 = jnp.float32
  src = edge_index[0]
  dst = edge_index[1]
  srcp = jnp.concatenate(
      [src, jnp.zeros((E_PAD - N_EDGES,), jnp.int32)]).reshape(NW, BLKS, EBLK)
  dstp = jnp.concatenate(
      [dst, jnp.full((E_PAD - N_EDGES,), DUMMY, jnp.int32)]
  ).reshape(NW, BLKS, EBLK)
  xi = jnp.concatenate(
      [jnp.squeeze(x, -1), jnp.zeros((X_PAD - N_NODES,), jnp.int32)]
  ).reshape(NW, XB, EBLK)
  zero_rows = jnp.zeros((ROWS_PER_TILE, SD), f32)

  w1i = nn_in_W1.T
  w2i = nn_in_W2.T
  w1o = nn_out_W1.T
  w2o = nn_out_W2.T
  b1i = nn_in_b1.reshape(1, SD)
  b2i = nn_in_b2.reshape(1, SD)
  b1o = nn_out_b1.reshape(1, SD)
  b2o = nn_out_b2.reshape(1, SD)
  gam = bn_gamma.reshape(1, SD)
  bet = bn_beta.reshape(1, SD)
  fc1T = fc1_W.T
  fc1b2 = fc1_b.reshape(1, SD)
  fc2T = jnp.zeros((SD, SD), f32).at[:, :N_CLASSES].set(fc2_W.T)
  fc2b2 = jnp.zeros((1, SD), f32).at[0, :N_CLASSES].set(fc2_b)
  batch3 = batch.reshape(NBLK, 1, NB)

  h = _embed(emb, xi)
  for _ in range(1 + HL):
    agg = _agg(h, srcp, dstp, zero_rows)
    h = _mlp_bn(h, agg, w1i, b1i, w2i, b2i, gam, bet)
  agg = _agg(h, srcp, dstp, zero_rows)
  out = _final(h, agg, w1o, b1o, w2o, b2o, batch3, fc1T, fc1b2, fc2T, fc2b2)
  return out[:, :N_CLASSES]


# speedup vs baseline: 1.1536x; 1.0118x over previous
import functools

import jax
import jax.numpy as jnp
from jax.experimental import pallas as pl
from jax.experimental.pallas import tpu as pltpu
from jax.experimental.pallas import tpu_sc as plsc

SD = 128
HL = 2
N_NODES = 10000
N_EDGES = 320000
N_GRAPHS = 64
VOCAB = 1340
N_CLASSES = 41

NC = 2
NS = 16
NW = NC * NS
EBLK = 128
BLKS = 80
E_PAD = NW * BLKS * EBLK
DUMMY = N_NODES
ROWS_PER_TILE = 632
ACC_ROWS = NS * ROWS_PER_TILE
XB = 3
X_PAD = NW * XB * EBLK

NB = 400
NBLK = N_NODES // NB

_vec_mesh = plsc.VectorSubcoreMesh(core_axis_name="core",
                                   subcore_axis_name="subcore")


def _embed(emb, xi):
  @functools.partial(
      pl.kernel,
      out_type=jax.ShapeDtypeStruct((X_PAD, SD), jnp.float32),
      mesh=_vec_mesh,
      scratch_types=[
          pltpu.VMEM((XB, EBLK), jnp.int32),
          pltpu.VMEM((EBLK, SD), jnp.float32),
      ],
  )
  def embed_kernel(emb_hbm, xi_hbm, out_hbm, xi_v, rows_v):
    c = jax.lax.axis_index("core")
    s = jax.lax.axis_index("subcore")
    wid = c * NS + s
    pltpu.sync_copy(xi_hbm.at[wid], xi_v)

    @pl.loop(0, XB)
    def _(j):
      pltpu.sync_copy(emb_hbm.at[xi_v.at[j]], rows_v)
      pltpu.sync_copy(rows_v, out_hbm.at[pl.ds(wid * XB * EBLK + j * EBLK,
                                               EBLK)])

  return embed_kernel(emb, xi)


def _agg(h, srcp, dstp, zero_rows):
  @functools.partial(
      pl.kernel,
      out_type=jax.ShapeDtypeStruct((NC, ACC_ROWS, SD), jnp.float32),
      mesh=_vec_mesh,
      scratch_types=[
          pltpu.VMEM_SHARED((ACC_ROWS, SD), jnp.float32),
          pltpu.VMEM((BLKS, EBLK), jnp.int32),
          pltpu.VMEM((BLKS, EBLK), jnp.int32),
          pltpu.VMEM((EBLK, SD), jnp.float32),
      ],
  )
  def agg_kernel(h_hbm, src_hbm, dst_hbm, zero_hbm, agg_hbm,
                 acc, src_v, dst_v, rows0):
    c = jax.lax.axis_index("core")
    s = jax.lax.axis_index("subcore")
    wid = c * NS + s
    pltpu.sync_copy(zero_hbm, acc.at[pl.ds(s * ROWS_PER_TILE, ROWS_PER_TILE)])
    pltpu.sync_copy(src_hbm.at[wid], src_v)
    pltpu.sync_copy(dst_hbm.at[wid], dst_v)
    plsc.subcore_barrier()

    @pl.loop(0, BLKS)
    def _(j):
      pltpu.sync_copy(h_hbm.at[src_v.at[j]], rows0)
      pltpu.sync_copy(rows0, acc.at[dst_v.at[j]], add=True)

    plsc.subcore_barrier()
    pltpu.sync_copy(acc.at[pl.ds(s * ROWS_PER_TILE, ROWS_PER_TILE)],
                    agg_hbm.at[c, pl.ds(s * ROWS_PER_TILE, ROWS_PER_TILE)])

  return agg_kernel(h, srcp, dstp, zero_rows)


def _mlp_body(h_ref, agg_ref, w1_ref, b1_ref, w2_ref, b2_ref,
              out_ref, st_ref):
  i = pl.program_id(0)
  z = h_ref[...] + agg_ref[0] + agg_ref[1]
  u = jnp.maximum(
      jnp.dot(z, w1_ref[...], preferred_element_type=jnp.float32)
      + b1_ref[...], 0.0)
  v = (jnp.dot(u, w2_ref[...], preferred_element_type=jnp.float32)
       + b2_ref[...])
  hout = jnp.maximum(v, 0.0)
  out_ref[...] = hout
  su = jnp.sum(hout, axis=0, keepdims=True)
  sq = jnp.sum(hout * hout, axis=0, keepdims=True)
  upd = jnp.concatenate([su, sq, jnp.zeros((6, SD), jnp.float32)], axis=0)

  @pl.when(i == 0)
  def _():
    st_ref[...] = upd

  @pl.when(i > 0)
  def _():
    st_ref[...] = st_ref[...] + upd


def _mlp(h, agg, w1, b1, w2, b2):
  return pl.pallas_call(
      _mlp_body,
      grid=(NBLK,),
      in_specs=[
          pl.BlockSpec((NB, SD), lambda i: (i, 0)),
          pl.BlockSpec((NC, NB, SD), lambda i: (0, i, 0)),
          pl.BlockSpec((SD, SD), lambda i: (0, 0)),
          pl.BlockSpec((1, SD), lambda i: (0, 0)),
          pl.BlockSpec((SD, SD), lambda i: (0, 0)),
          pl.BlockSpec((1, SD), lambda i: (0, 0)),
      ],
      out_specs=[
          pl.BlockSpec((NB, SD), lambda i: (i, 0)),
          pl.BlockSpec((8, SD), lambda i: (0, 0)),
      ],
      out_shape=[
          jax.ShapeDtypeStruct((N_NODES, SD), jnp.float32),
          jax.ShapeDtypeStruct((8, SD), jnp.float32),
      ],
  )(h, agg, w1, b1, w2, b2)


def _norm_body(v_ref, st_ref, g_ref, b_ref, out_ref):
  inv_n = 1.0 / N_NODES
  mean = st_ref[0:1, :] * inv_n
  ex2 = st_ref[1:2, :] * inv_n
  var = ex2 - mean * mean
  a = g_ref[...] * jax.lax.rsqrt(var + 1e-5)
  b = b_ref[...] - mean * a
  out_ref[...] = v_ref[...] * a + b


def _norm(v, st, gamma, beta):
  return pl.pallas_call(
      _norm_body,
      grid=(NBLK,),
      in_specs=[
          pl.BlockSpec((NB, SD), lambda i: (i, 0)),
          pl.BlockSpec((8, SD), lambda i: (0, 0)),
          pl.BlockSpec((1, SD), lambda i: (0, 0)),
          pl.BlockSpec((1, SD), lambda i: (0, 0)),
      ],
      out_specs=pl.BlockSpec((NB, SD), lambda i: (i, 0)),
      out_shape=jax.ShapeDtypeStruct((N_NODES, SD), jnp.float32),
  )(v, st, gamma, beta)


def _final_body(h_ref, agg_ref, w1_ref, b1_ref, w2_ref, b2_ref, batch_ref,
                fc1w_ref, fc1b_ref, fc2w_ref, fc2b_ref, out_ref, acc_ref):
  i = pl.program_id(0)
  z = h_ref[...] + agg_ref[0] + agg_ref[1]
  u = jnp.maximum(
      jnp.dot(z, w1_ref[...], preferred_element_type=jnp.float32)
      + b1_ref[...], 0.0)
  v = (jnp.dot(u, w2_ref[...], preferred_element_type=jnp.float32)
       + b2_ref[...])
  hout = jnp.maximum(v, 0.0)
  bids = batch_ref[0, 0, :]
  rows = jax.lax.broadcasted_iota(jnp.int32, (N_GRAPHS, NB), 0)
  onehot = (rows == bids[None, :]).astype(jnp.float32)
  contrib = jnp.dot(onehot, hout, preferred_element_type=jnp.float32)

  @pl.when(i == 0)
  def _():
    acc_ref[...] = contrib

  @pl.when(i > 0)
  def _():
    acc_ref[...] = acc_ref[...] + contrib

  @pl.when(i == NBLK - 1)
  def _():
    g = jnp.maximum(
        jnp.dot(acc_ref[...], fc1w_ref[...],
                preferred_element_type=jnp.float32) + fc1b_ref[...], 0.0)
    out_ref[...] = (jnp.dot(g, fc2w_ref[...],
                            preferred_element_type=jnp.float32)
                    + fc2b_ref[...])


def _final(h, agg, w1, b1, w2, b2, batch3, fc1w, fc1b, fc2w, fc2b):
  return pl.pallas_call(
      _final_body,
      grid=(NBLK,),
      in_specs=[
          pl.BlockSpec((NB, SD), lambda i: (i, 0)),
          pl.BlockSpec((NC, NB, SD), lambda i: (0, i, 0)),
          pl.BlockSpec((SD, SD), lambda i: (0, 0)),
          pl.BlockSpec((1, SD), lambda i: (0, 0)),
          pl.BlockSpec((SD, SD), lambda i: (0, 0)),
          pl.BlockSpec((1, SD), lambda i: (0, 0)),
          pl.BlockSpec((1, 1, NB), lambda i: (i, 0, 0)),
          pl.BlockSpec((SD, SD), lambda i: (0, 0)),
          pl.BlockSpec((1, SD), lambda i: (0, 0)),
          pl.BlockSpec((SD, SD), lambda i: (0, 0)),
          pl.BlockSpec((1, SD), lambda i: (0, 0)),
      ],
      out_specs=pl.BlockSpec((N_GRAPHS, SD), lambda i: (0, 0)),
      out_shape=jax.ShapeDtypeStruct((N_GRAPHS, SD), jnp.float32),
      scratch_shapes=[pltpu.VMEM((N_GRAPHS, SD), jnp.float32)],
  )(h, agg, w1, b1, w2, b2, batch3, fc1w, fc1b, fc2w, fc2b)


def kernel(x, edge_index, batch, emb, nn_in_W1, nn_in_b1, nn_in_W2, nn_in_b2,
           bn_gamma, bn_beta, nn_out_W1, nn_out_b1, nn_out_W2, nn_out_b2,
           fc1_W, fc1_b, fc2_W, fc2_b):
  f32 = jnp.float32
  src = edge_index[0]
  dst = edge_index[1]
  srcp = jnp.concatenate(
      [src, jnp.zeros((E_PAD - N_EDGES,), jnp.int32)]).reshape(NW, BLKS, EBLK)
  dstp = jnp.concatenate(
      [dst, jnp.full((E_PAD - N_EDGES,), DUMMY, jnp.int32)]
  ).reshape(NW, BLKS, EBLK)
  xi = jnp.concatenate(
      [jnp.squeeze(x, -1), jnp.zeros((X_PAD - N_NODES,), jnp.int32)]
  ).reshape(NW, XB, EBLK)
  zero_rows = jnp.zeros((ROWS_PER_TILE, SD), f32)

  w1i = nn_in_W1.T
  w2i = nn_in_W2.T
  w1o = nn_out_W1.T
  w2o = nn_out_W2.T
  b1i = nn_in_b1.reshape(1, SD)
  b2i = nn_in_b2.reshape(1, SD)
  b1o = nn_out_b1.reshape(1, SD)
  b2o = nn_out_b2.reshape(1, SD)
  gam = bn_gamma.reshape(1, SD)
  bet = bn_beta.reshape(1, SD)
  fc1T = fc1_W.T
  fc1b2 = fc1_b.reshape(1, SD)
  fc2T = jnp.zeros((SD, SD), f32).at[:, :N_CLASSES].set(fc2_W.T)
  fc2b2 = jnp.zeros((1, SD), f32).at[0, :N_CLASSES].set(fc2_b)
  batch3 = batch.reshape(NBLK, 1, NB)

  h = _embed(emb, xi)
  for _ in range(1 + HL):
    agg = _agg(h, srcp, dstp, zero_rows)
    v, st = _mlp(h, agg, w1i, b1i, w2i, b2i)
    h = _norm(v, st, gam, bet)
  agg = _agg(h, srcp, dstp, zero_rows)
  out = _final(h, agg, w1o, b1o, w2o, b2o, batch3, fc1T, fc1b2, fc2T, fc2b2)
  return out[:, :N_CLASSES]


# speedup vs baseline: 1.6783x; 1.4549x over previous
import functools

import jax
import jax.numpy as jnp
from jax.experimental import pallas as pl
from jax.experimental.pallas import tpu as pltpu
from jax.experimental.pallas import tpu_sc as plsc

SD = 128
HL = 2
N_NODES = 10000
N_EDGES = 320000
N_GRAPHS = 64
VOCAB = 1340
N_CLASSES = 41

NC = 2
NS = 16
NW = NC * NS
EBLK = 128
BLKS = 79
E_PAD = NW * BLKS * EBLK
DUMMY = N_NODES
ROWS_PER_TILE = 632
ACC_ROWS = NS * ROWS_PER_TILE
XB = 3
X_PAD = NW * XB * EBLK

NB = 400
NBLK = N_NODES // NB

_vec_mesh = plsc.VectorSubcoreMesh(core_axis_name="core",
                                   subcore_axis_name="subcore")


def _embed(emb, xi):
  @functools.partial(
      pl.kernel,
      out_type=jax.ShapeDtypeStruct((X_PAD, SD), jnp.float32),
      mesh=_vec_mesh,
      scratch_types=[
          pltpu.VMEM((XB, EBLK), jnp.int32),
          pltpu.VMEM((EBLK, SD), jnp.float32),
      ],
  )
  def embed_kernel(emb_hbm, xi_hbm, out_hbm, xi_v, rows_v):
    c = jax.lax.axis_index("core")
    s = jax.lax.axis_index("subcore")
    wid = c * NS + s
    pltpu.sync_copy(xi_hbm.at[wid], xi_v)

    @pl.loop(0, XB)
    def _(j):
      pltpu.sync_copy(emb_hbm.at[xi_v.at[j]], rows_v)
      pltpu.sync_copy(rows_v, out_hbm.at[pl.ds(wid * XB * EBLK + j * EBLK,
                                               EBLK)])

  return embed_kernel(emb, xi)


def _agg(h, srcp, dstp, zero_rows):
  @functools.partial(
      pl.kernel,
      out_type=jax.ShapeDtypeStruct((NC, ACC_ROWS, SD), jnp.float32),
      mesh=_vec_mesh,
      scratch_types=[
          pltpu.VMEM_SHARED((ACC_ROWS, SD), jnp.float32),
          pltpu.VMEM((BLKS, EBLK), jnp.int32),
          pltpu.VMEM((BLKS, EBLK), jnp.int32),
          pltpu.VMEM((EBLK, SD), jnp.float32),
      ],
  )
  def agg_kernel(h_hbm, src_hbm, dst_hbm, zero_hbm, agg_hbm,
                 acc, src_v, dst_v, rows0):
    c = jax.lax.axis_index("core")
    s = jax.lax.axis_index("subcore")
    wid = c * NS + s
    pltpu.sync_copy(zero_hbm, acc.at[pl.ds(s * ROWS_PER_TILE, ROWS_PER_TILE)])
    pltpu.sync_copy(src_hbm.at[wid], src_v)
    pltpu.sync_copy(dst_hbm.at[wid], dst_v)
    plsc.subcore_barrier()

    @pl.loop(0, BLKS)
    def _(j):
      pltpu.sync_copy(h_hbm.at[src_v.at[j]], rows0)
      pltpu.sync_copy(rows0, acc.at[dst_v.at[j]], add=True)

    plsc.subcore_barrier()
    pltpu.sync_copy(acc.at[pl.ds(s * ROWS_PER_TILE, ROWS_PER_TILE)],
                    agg_hbm.at[c, pl.ds(s * ROWS_PER_TILE, ROWS_PER_TILE)])

  return agg_kernel(h, srcp, dstp, zero_rows)


def _mlp_body(h_ref, agg_ref, w1_ref, b1_ref, w2_ref, b2_ref,
              out_ref, st_ref):
  i = pl.program_id(0)
  z = h_ref[...] + agg_ref[0] + agg_ref[1]
  u = jnp.maximum(
      jnp.dot(z, w1_ref[...], preferred_element_type=jnp.float32)
      + b1_ref[...], 0.0)
  v = (jnp.dot(u, w2_ref[...], preferred_element_type=jnp.float32)
       + b2_ref[...])
  hout = jnp.maximum(v, 0.0)
  out_ref[...] = hout
  su = jnp.sum(hout, axis=0, keepdims=True)
  sq = jnp.sum(hout * hout, axis=0, keepdims=True)
  upd = jnp.concatenate([su, sq, jnp.zeros((6, SD), jnp.float32)], axis=0)

  @pl.when(i == 0)
  def _():
    st_ref[...] = upd

  @pl.when(i > 0)
  def _():
    st_ref[...] = st_ref[...] + upd


def _mlp(h, agg, w1, b1, w2, b2):
  return pl.pallas_call(
      _mlp_body,
      grid=(NBLK,),
      in_specs=[
          pl.BlockSpec((NB, SD), lambda i: (i, 0)),
          pl.BlockSpec((NC, NB, SD), lambda i: (0, i, 0)),
          pl.BlockSpec((SD, SD), lambda i: (0, 0)),
          pl.BlockSpec((1, SD), lambda i: (0, 0)),
          pl.BlockSpec((SD, SD), lambda i: (0, 0)),
          pl.BlockSpec((1, SD), lambda i: (0, 0)),
      ],
      out_specs=[
          pl.BlockSpec((NB, SD), lambda i: (i, 0)),
          pl.BlockSpec((8, SD), lambda i: (0, 0)),
      ],
      out_shape=[
          jax.ShapeDtypeStruct((N_NODES, SD), jnp.float32),
          jax.ShapeDtypeStruct((8, SD), jnp.float32),
      ],
  )(h, agg, w1, b1, w2, b2)


def _norm_body(v_ref, st_ref, g_ref, b_ref, out_ref):
  inv_n = 1.0 / N_NODES
  mean = st_ref[0:1, :] * inv_n
  ex2 = st_ref[1:2, :] * inv_n
  var = ex2 - mean * mean
  a = g_ref[...] * jax.lax.rsqrt(var + 1e-5)
  b = b_ref[...] - mean * a
  out_ref[...] = v_ref[...] * a + b


def _norm(v, st, gamma, beta):
  return pl.pallas_call(
      _norm_body,
      grid=(NBLK,),
      in_specs=[
          pl.BlockSpec((NB, SD), lambda i: (i, 0)),
          pl.BlockSpec((8, SD), lambda i: (0, 0)),
          pl.BlockSpec((1, SD), lambda i: (0, 0)),
          pl.BlockSpec((1, SD), lambda i: (0, 0)),
      ],
      out_specs=pl.BlockSpec((NB, SD), lambda i: (i, 0)),
      out_shape=jax.ShapeDtypeStruct((N_NODES, SD), jnp.float32),
  )(v, st, gamma, beta)


def _final_body(h_ref, agg_ref, w1_ref, b1_ref, w2_ref, b2_ref, batch_ref,
                fc1w_ref, fc1b_ref, fc2w_ref, fc2b_ref, out_ref, acc_ref):
  i = pl.program_id(0)
  z = h_ref[...] + agg_ref[0] + agg_ref[1]
  u = jnp.maximum(
      jnp.dot(z, w1_ref[...], preferred_element_type=jnp.float32)
      + b1_ref[...], 0.0)
  v = (jnp.dot(u, w2_ref[...], preferred_element_type=jnp.float32)
       + b2_ref[...])
  hout = jnp.maximum(v, 0.0)
  bids = batch_ref[0, 0, :]
  rows = jax.lax.broadcasted_iota(jnp.int32, (N_GRAPHS, NB), 0)
  onehot = (rows == bids[None, :]).astype(jnp.float32)
  contrib = jnp.dot(onehot, hout, preferred_element_type=jnp.float32)

  @pl.when(i == 0)
  def _():
    acc_ref[...] = contrib

  @pl.when(i > 0)
  def _():
    acc_ref[...] = acc_ref[...] + contrib

  @pl.when(i == NBLK - 1)
  def _():
    g = jnp.maximum(
        jnp.dot(acc_ref[...], fc1w_ref[...],
                preferred_element_type=jnp.float32) + fc1b_ref[...], 0.0)
    out_ref[...] = (jnp.dot(g, fc2w_ref[...],
                            preferred_element_type=jnp.float32)
                    + fc2b_ref[...])


def _final(h, agg, w1, b1, w2, b2, batch3, fc1w, fc1b, fc2w, fc2b):
  return pl.pallas_call(
      _final_body,
      grid=(NBLK,),
      in_specs=[
          pl.BlockSpec((NB, SD), lambda i: (i, 0)),
          pl.BlockSpec((NC, NB, SD), lambda i: (0, i, 0)),
          pl.BlockSpec((SD, SD), lambda i: (0, 0)),
          pl.BlockSpec((1, SD), lambda i: (0, 0)),
          pl.BlockSpec((SD, SD), lambda i: (0, 0)),
          pl.BlockSpec((1, SD), lambda i: (0, 0)),
          pl.BlockSpec((1, 1, NB), lambda i: (i, 0, 0)),
          pl.BlockSpec((SD, SD), lambda i: (0, 0)),
          pl.BlockSpec((1, SD), lambda i: (0, 0)),
          pl.BlockSpec((SD, SD), lambda i: (0, 0)),
          pl.BlockSpec((1, SD), lambda i: (0, 0)),
      ],
      out_specs=pl.BlockSpec((N_GRAPHS, SD), lambda i: (0, 0)),
      out_shape=jax.ShapeDtypeStruct((N_GRAPHS, SD), jnp.float32),
      scratch_shapes=[pltpu.VMEM((N_GRAPHS, SD), jnp.float32)],
  )(h, agg, w1, b1, w2, b2, batch3, fc1w, fc1b, fc2w, fc2b)


def kernel(x, edge_index, batch, emb, nn_in_W1, nn_in_b1, nn_in_W2, nn_in_b2,
           bn_gamma, bn_beta, nn_out_W1, nn_out_b1, nn_out_W2, nn_out_b2,
           fc1_W, fc1_b, fc2_W, fc2_b):
  f32 = jnp.float32
  src = edge_index[0]
  dst = edge_index[1]
  srcp = jnp.concatenate(
      [src, jnp.zeros((E_PAD - N_EDGES,), jnp.int32)]).reshape(NW, BLKS, EBLK)
  dstp = jnp.concatenate(
      [dst, jnp.full((E_PAD - N_EDGES,), DUMMY, jnp.int32)]
  ).reshape(NW, BLKS, EBLK)
  xi = jnp.concatenate(
      [jnp.squeeze(x, -1), jnp.zeros((X_PAD - N_NODES,), jnp.int32)]
  ).reshape(NW, XB, EBLK)
  zero_rows = jnp.zeros((ROWS_PER_TILE, SD), f32)

  w1i = nn_in_W1.T
  w2i = nn_in_W2.T
  w1o = nn_out_W1.T
  w2o = nn_out_W2.T
  b1i = nn_in_b1.reshape(1, SD)
  b2i = nn_in_b2.reshape(1, SD)
  b1o = nn_out_b1.reshape(1, SD)
  b2o = nn_out_b2.reshape(1, SD)
  gam = bn_gamma.reshape(1, SD)
  bet = bn_beta.reshape(1, SD)
  fc1T = fc1_W.T
  fc1b2 = fc1_b.reshape(1, SD)
  fc2T = jnp.zeros((SD, SD), f32).at[:, :N_CLASSES].set(fc2_W.T)
  fc2b2 = jnp.zeros((1, SD), f32).at[0, :N_CLASSES].set(fc2_b)
  batch3 = batch.reshape(NBLK, 1, NB)

  h = _embed(emb, xi)
  for _ in range(1 + HL):
    agg = _agg(h, srcp, dstp, zero_rows)
    v, st = _mlp(h, agg, w1i, b1i, w2i, b2i)
    h = _norm(v, st, gam, bet)
  agg = _agg(h, srcp, dstp, zero_rows)
  out = _final(h, agg, w1o, b1o, w2o, b2o, batch3, fc1T, fc1b2, fc2T, fc2b2)
  return out[:, :N_CLASSES]


# speedup vs baseline: 2.9173x; 1.7383x over previous
import functools

import jax
import jax.numpy as jnp
from jax.experimental import pallas as pl
from jax.experimental.pallas import tpu as pltpu
from jax.experimental.pallas import tpu_sc as plsc

SD = 128
HL = 2
N_NODES = 10000
N_EDGES = 320000
N_GRAPHS = 64
VOCAB = 1340
N_CLASSES = 41

NC = 2
NS = 16
NW = NC * NS
EBLK = 128
BLKS = 80
E_PAD = NW * BLKS * EBLK
DUMMY = N_NODES
ROWS_PER_TILE = 632
ACC_ROWS = NS * ROWS_PER_TILE
XB = 3
X_PAD = NW * XB * EBLK

NB = 400
NBLK = N_NODES // NB

_vec_mesh = plsc.VectorSubcoreMesh(core_axis_name="core",
                                   subcore_axis_name="subcore")


def _embed(emb, xi):
  @functools.partial(
      pl.kernel,
      out_type=jax.ShapeDtypeStruct((X_PAD, SD), jnp.float32),
      mesh=_vec_mesh,
      scratch_types=[
          pltpu.VMEM((XB, EBLK), jnp.int32),
          pltpu.VMEM((EBLK, SD), jnp.float32),
      ],
  )
  def embed_kernel(emb_hbm, xi_hbm, out_hbm, xi_v, rows_v):
    c = jax.lax.axis_index("core")
    s = jax.lax.axis_index("subcore")
    wid = c * NS + s
    pltpu.sync_copy(xi_hbm.at[wid], xi_v)

    @pl.loop(0, XB)
    def _(j):
      pltpu.sync_copy(emb_hbm.at[xi_v.at[j]], rows_v)
      pltpu.sync_copy(rows_v, out_hbm.at[pl.ds(wid * XB * EBLK + j * EBLK,
                                               EBLK)])

  return embed_kernel(emb, xi)


def _agg(h, srcp, dstp, zero_rows):
  @functools.partial(
      pl.kernel,
      out_type=jax.ShapeDtypeStruct((NC, ACC_ROWS, SD), jnp.float32),
      mesh=_vec_mesh,
      scratch_types=[
          pltpu.VMEM_SHARED((ACC_ROWS, SD), jnp.float32),
          pltpu.VMEM((BLKS, EBLK), jnp.int32),
          pltpu.VMEM((BLKS, EBLK), jnp.int32),
          pltpu.VMEM((EBLK, SD), jnp.float32),
      ],
  )
  def agg_kernel(h_hbm, src_hbm, dst_hbm, zero_hbm, agg_hbm,
                 acc, src_v, dst_v, rows0):
    c = jax.lax.axis_index("core")
    s = jax.lax.axis_index("subcore")
    wid = c * NS + s
    pltpu.sync_copy(zero_hbm, acc.at[pl.ds(s * ROWS_PER_TILE, ROWS_PER_TILE)])
    pltpu.sync_copy(src_hbm.at[wid], src_v)
    pltpu.sync_copy(dst_hbm.at[wid], dst_v)
    plsc.subcore_barrier()

    @pl.loop(0, BLKS)
    def _(j):
      pltpu.sync_copy(h_hbm.at[src_v.at[j]], rows0)
      pltpu.sync_copy(rows0, acc.at[dst_v.at[j]], add=True)

    plsc.subcore_barrier()
    pltpu.sync_copy(acc.at[pl.ds(s * ROWS_PER_TILE, ROWS_PER_TILE)],
                    agg_hbm.at[c, pl.ds(s * ROWS_PER_TILE, ROWS_PER_TILE)])

  return agg_kernel(h, srcp, dstp, zero_rows)


def _mlp_body(h_ref, agg_ref, w1_ref, b1_ref, w2_ref, b2_ref,
              out_ref, st_ref):
  i = pl.program_id(0)
  z = h_ref[...] + agg_ref[0] + agg_ref[1]
  u = jnp.maximum(
      jnp.dot(z, w1_ref[...], preferred_element_type=jnp.float32)
      + b1_ref[...], 0.0)
  v = (jnp.dot(u, w2_ref[...], preferred_element_type=jnp.float32)
       + b2_ref[...])
  hout = jnp.maximum(v, 0.0)
  out_ref[...] = hout
  su = jnp.sum(hout, axis=0, keepdims=True)
  sq = jnp.sum(hout * hout, axis=0, keepdims=True)
  upd = jnp.concatenate([su, sq, jnp.zeros((6, SD), jnp.float32)], axis=0)

  @pl.when(i == 0)
  def _():
    st_ref[...] = upd

  @pl.when(i > 0)
  def _():
    st_ref[...] = st_ref[...] + upd


def _mlp(h, agg, w1, b1, w2, b2):
  return pl.pallas_call(
      _mlp_body,
      grid=(NBLK,),
      in_specs=[
          pl.BlockSpec((NB, SD), lambda i: (i, 0)),
          pl.BlockSpec((NC, NB, SD), lambda i: (0, i, 0)),
          pl.BlockSpec((SD, SD), lambda i: (0, 0)),
          pl.BlockSpec((1, SD), lambda i: (0, 0)),
          pl.BlockSpec((SD, SD), lambda i: (0, 0)),
          pl.BlockSpec((1, SD), lambda i: (0, 0)),
      ],
      out_specs=[
          pl.BlockSpec((NB, SD), lambda i: (i, 0)),
          pl.BlockSpec((8, SD), lambda i: (0, 0)),
      ],
      out_shape=[
          jax.ShapeDtypeStruct((N_NODES, SD), jnp.float32),
          jax.ShapeDtypeStruct((8, SD), jnp.float32),
      ],
  )(h, agg, w1, b1, w2, b2)


def _norm_body(v_ref, st_ref, g_ref, b_ref, out_ref):
  inv_n = 1.0 / N_NODES
  mean = st_ref[0:1, :] * inv_n
  ex2 = st_ref[1:2, :] * inv_n
  var = ex2 - mean * mean
  a = g_ref[...] * jax.lax.rsqrt(var + 1e-5)
  b = b_ref[...] - mean * a
  out_ref[...] = v_ref[...] * a + b


def _norm(v, st, gamma, beta):
  return pl.pallas_call(
      _norm_body,
      grid=(NBLK,),
      in_specs=[
          pl.BlockSpec((NB, SD), lambda i: (i, 0)),
          pl.BlockSpec((8, SD), lambda i: (0, 0)),
          pl.BlockSpec((1, SD), lambda i: (0, 0)),
          pl.BlockSpec((1, SD), lambda i: (0, 0)),
      ],
      out_specs=pl.BlockSpec((NB, SD), lambda i: (i, 0)),
      out_shape=jax.ShapeDtypeStruct((N_NODES, SD), jnp.float32),
  )(v, st, gamma, beta)


def _final_body(h_ref, agg_ref, w1_ref, b1_ref, w2_ref, b2_ref, batch_ref,
                fc1w_ref, fc1b_ref, fc2w_ref, fc2b_ref, out_ref, acc_ref):
  i = pl.program_id(0)
  z = h_ref[...] + agg_ref[0] + agg_ref[1]
  u = jnp.maximum(
      jnp.dot(z, w1_ref[...], preferred_element_type=jnp.float32)
      + b1_ref[...], 0.0)
  v = (jnp.dot(u, w2_ref[...], preferred_element_type=jnp.float32)
       + b2_ref[...])
  hout = jnp.maximum(v, 0.0)
  bids = batch_ref[0, 0, :]
  rows = jax.lax.broadcasted_iota(jnp.int32, (N_GRAPHS, NB), 0)
  onehot = (rows == bids[None, :]).astype(jnp.float32)
  contrib = jnp.dot(onehot, hout, preferred_element_type=jnp.float32)

  @pl.when(i == 0)
  def _():
    acc_ref[...] = contrib

  @pl.when(i > 0)
  def _():
    acc_ref[...] = acc_ref[...] + contrib

  @pl.when(i == NBLK - 1)
  def _():
    g = jnp.maximum(
        jnp.dot(acc_ref[...], fc1w_ref[...],
                preferred_element_type=jnp.float32) + fc1b_ref[...], 0.0)
    out_ref[...] = (jnp.dot(g, fc2w_ref[...],
                            preferred_element_type=jnp.float32)
                    + fc2b_ref[...])


def _final(h, agg, w1, b1, w2, b2, batch3, fc1w, fc1b, fc2w, fc2b):
  return pl.pallas_call(
      _final_body,
      grid=(NBLK,),
      in_specs=[
          pl.BlockSpec((NB, SD), lambda i: (i, 0)),
          pl.BlockSpec((NC, NB, SD), lambda i: (0, i, 0)),
          pl.BlockSpec((SD, SD), lambda i: (0, 0)),
          pl.BlockSpec((1, SD), lambda i: (0, 0)),
          pl.BlockSpec((SD, SD), lambda i: (0, 0)),
          pl.BlockSpec((1, SD), lambda i: (0, 0)),
          pl.BlockSpec((1, 1, NB), lambda i: (i, 0, 0)),
          pl.BlockSpec((SD, SD), lambda i: (0, 0)),
          pl.BlockSpec((1, SD), lambda i: (0, 0)),
          pl.BlockSpec((SD, SD), lambda i: (0, 0)),
          pl.BlockSpec((1, SD), lambda i: (0, 0)),
      ],
      out_specs=pl.BlockSpec((N_GRAPHS, SD), lambda i: (0, 0)),
      out_shape=jax.ShapeDtypeStruct((N_GRAPHS, SD), jnp.float32),
      scratch_shapes=[pltpu.VMEM((N_GRAPHS, SD), jnp.float32)],
  )(h, agg, w1, b1, w2, b2, batch3, fc1w, fc1b, fc2w, fc2b)


def kernel(x, edge_index, batch, emb, nn_in_W1, nn_in_b1, nn_in_W2, nn_in_b2,
           bn_gamma, bn_beta, nn_out_W1, nn_out_b1, nn_out_W2, nn_out_b2,
           fc1_W, fc1_b, fc2_W, fc2_b):
  f32 = jnp.float32
  src = edge_index[0]
  dst = edge_index[1]
  pad_iota = jnp.arange(E_PAD - N_EDGES, dtype=jnp.int32)
  srcp = jnp.concatenate(
      [src, pad_iota % N_NODES]).reshape(NW, BLKS, EBLK)
  dstp = jnp.concatenate(
      [dst, DUMMY + pad_iota % (ACC_ROWS - N_NODES)]
  ).reshape(NW, BLKS, EBLK)
  xi = jnp.concatenate(
      [jnp.squeeze(x, -1),
       jnp.arange(X_PAD - N_NODES, dtype=jnp.int32) % VOCAB]
  ).reshape(NW, XB, EBLK)
  zero_rows = jnp.zeros((ROWS_PER_TILE, SD), f32)

  w1i = nn_in_W1.T
  w2i = nn_in_W2.T
  w1o = nn_out_W1.T
  w2o = nn_out_W2.T
  b1i = nn_in_b1.reshape(1, SD)
  b2i = nn_in_b2.reshape(1, SD)
  b1o = nn_out_b1.reshape(1, SD)
  b2o = nn_out_b2.reshape(1, SD)
  gam = bn_gamma.reshape(1, SD)
  bet = bn_beta.reshape(1, SD)
  fc1T = fc1_W.T
  fc1b2 = fc1_b.reshape(1, SD)
  fc2T = jnp.zeros((SD, SD), f32).at[:, :N_CLASSES].set(fc2_W.T)
  fc2b2 = jnp.zeros((1, SD), f32).at[0, :N_CLASSES].set(fc2_b)
  batch3 = batch.reshape(NBLK, 1, NB)

  h = _embed(emb, xi)
  for _ in range(1 + HL):
    agg = _agg(h, srcp, dstp, zero_rows)
    v, st = _mlp(h, agg, w1i, b1i, w2i, b2i)
    h = _norm(v, st, gam, bet)
  agg = _agg(h, srcp, dstp, zero_rows)
  out = _final(h, agg, w1o, b1o, w2o, b2o, batch3, fc1T, fc1b2, fc2T, fc2b2)
  return out[:, :N_CLASSES]


# speedup vs baseline: 3.9783x; 1.3637x over previous
import functools

import jax
import jax.numpy as jnp
from jax.experimental import pallas as pl
from jax.experimental.pallas import tpu as pltpu
from jax.experimental.pallas import tpu_sc as plsc

SD = 128
HL = 2
N_NODES = 10000
N_EDGES = 320000
N_GRAPHS = 64
VOCAB = 1340
N_CLASSES = 41

NC = 2
NS = 16
NW = NC * NS
EBLK = 128
BLKS = 80
CHB = 16
NCH = BLKS // CHB
E_PAD = NW * BLKS * EBLK
DUMMY = N_NODES
ROWS_PER_TILE = 632
ACC_ROWS = NS * ROWS_PER_TILE
XB = 3
X_PAD = NW * XB * EBLK

NB = 400
NBLK = N_NODES // NB

_vec_mesh = plsc.VectorSubcoreMesh(core_axis_name="core",
                                   subcore_axis_name="subcore")


def _embed(emb, xi):
  @functools.partial(
      pl.kernel,
      out_type=jax.ShapeDtypeStruct((X_PAD, SD), jnp.float32),
      mesh=_vec_mesh,
      scratch_types=[
          pltpu.VMEM((XB, EBLK), jnp.int32),
          pltpu.VMEM((EBLK, SD), jnp.float32),
      ],
  )
  def embed_kernel(emb_hbm, xi_hbm, out_hbm, xi_v, rows_v):
    c = jax.lax.axis_index("core")
    s = jax.lax.axis_index("subcore")
    wid = c * NS + s
    pltpu.sync_copy(xi_hbm.at[wid], xi_v)

    @pl.loop(0, XB)
    def _(j):
      pltpu.sync_copy(emb_hbm.at[xi_v.at[j]], rows_v)
      pltpu.sync_copy(rows_v, out_hbm.at[pl.ds(wid * XB * EBLK + j * EBLK,
                                               EBLK)])

  return embed_kernel(emb, xi)


def _agg(h, srcp, dstp, zero_rows):
  @functools.partial(
      pl.kernel,
      out_type=jax.ShapeDtypeStruct((NC, ACC_ROWS, SD), jnp.float32),
      mesh=_vec_mesh,
      scratch_types=[
          pltpu.VMEM_SHARED((ACC_ROWS, SD), jnp.float32),
          pltpu.VMEM((CHB, EBLK), jnp.int32),
          pltpu.VMEM((CHB, EBLK), jnp.int32),
          pltpu.VMEM((EBLK, SD), jnp.float32),
          pltpu.VMEM((EBLK, SD), jnp.float32),
          pltpu.SemaphoreType.DMA,
          pltpu.SemaphoreType.DMA,
      ],
  )
  def agg_kernel(h_hbm, src_hbm, dst_hbm, zero_hbm, agg_hbm,
                 acc, sidx, didx, rows0, rows1, sem_g0, sem_g1):
    c = jax.lax.axis_index("core")
    s = jax.lax.axis_index("subcore")
    wid = c * NS + s
    rows = (rows0, rows1)
    sem_g = (sem_g0, sem_g1)
    pltpu.sync_copy(zero_hbm, acc.at[pl.ds(s * ROWS_PER_TILE, ROWS_PER_TILE)])
    plsc.subcore_barrier()

    @pl.loop(0, NCH)
    def _(ck):
      pltpu.sync_copy(src_hbm.at[wid, pl.ds(ck * CHB, CHB)], sidx)
      pltpu.sync_copy(dst_hbm.at[wid, pl.ds(ck * CHB, CHB)], didx)
      d = pltpu.async_copy(h_hbm.at[sidx.at[0]], rows[0], sem_g0)
      for b in range(CHB):
        d_next = None
        if b + 1 < CHB:
          d_next = pltpu.async_copy(h_hbm.at[sidx.at[b + 1]],
                                    rows[(b + 1) % 2], sem_g[(b + 1) % 2])
        d.wait()
        pltpu.sync_copy(rows[b % 2], acc.at[didx.at[b]], add=True)
        d = d_next

    plsc.subcore_barrier()
    pltpu.sync_copy(acc.at[pl.ds(s * ROWS_PER_TILE, ROWS_PER_TILE)],
                    agg_hbm.at[c, pl.ds(s * ROWS_PER_TILE, ROWS_PER_TILE)])

  return agg_kernel(h, srcp, dstp, zero_rows)


def _mlp_body(h_ref, agg_ref, w1_ref, b1_ref, w2_ref, b2_ref,
              out_ref, st_ref):
  i = pl.program_id(0)
  z = h_ref[...] + agg_ref[0] + agg_ref[1]
  u = jnp.maximum(
      jnp.dot(z, w1_ref[...], preferred_element_type=jnp.float32)
      + b1_ref[...], 0.0)
  v = (jnp.dot(u, w2_ref[...], preferred_element_type=jnp.float32)
       + b2_ref[...])
  hout = jnp.maximum(v, 0.0)
  out_ref[...] = hout
  su = jnp.sum(hout, axis=0, keepdims=True)
  sq = jnp.sum(hout * hout, axis=0, keepdims=True)
  upd = jnp.concatenate([su, sq, jnp.zeros((6, SD), jnp.float32)], axis=0)

  @pl.when(i == 0)
  def _():
    st_ref[...] = upd

  @pl.when(i > 0)
  def _():
    st_ref[...] = st_ref[...] + upd


def _mlp(h, agg, w1, b1, w2, b2):
  return pl.pallas_call(
      _mlp_body,
      grid=(NBLK,),
      in_specs=[
          pl.BlockSpec((NB, SD), lambda i: (i, 0)),
          pl.BlockSpec((NC, NB, SD), lambda i: (0, i, 0)),
          pl.BlockSpec((SD, SD), lambda i: (0, 0)),
          pl.BlockSpec((1, SD), lambda i: (0, 0)),
          pl.BlockSpec((SD, SD), lambda i: (0, 0)),
          pl.BlockSpec((1, SD), lambda i: (0, 0)),
      ],
      out_specs=[
          pl.BlockSpec((NB, SD), lambda i: (i, 0)),
          pl.BlockSpec((8, SD), lambda i: (0, 0)),
      ],
      out_shape=[
          jax.ShapeDtypeStruct((N_NODES, SD), jnp.float32),
          jax.ShapeDtypeStruct((8, SD), jnp.float32),
      ],
  )(h, agg, w1, b1, w2, b2)


def _norm_body(v_ref, st_ref, g_ref, b_ref, out_ref):
  inv_n = 1.0 / N_NODES
  mean = st_ref[0:1, :] * inv_n
  ex2 = st_ref[1:2, :] * inv_n
  var = ex2 - mean * mean
  a = g_ref[...] * jax.lax.rsqrt(var + 1e-5)
  b = b_ref[...] - mean * a
  out_ref[...] = v_ref[...] * a + b


def _norm(v, st, gamma, beta):
  return pl.pallas_call(
      _norm_body,
      grid=(NBLK,),
      in_specs=[
          pl.BlockSpec((NB, SD), lambda i: (i, 0)),
          pl.BlockSpec((8, SD), lambda i: (0, 0)),
          pl.BlockSpec((1, SD), lambda i: (0, 0)),
          pl.BlockSpec((1, SD), lambda i: (0, 0)),
      ],
      out_specs=pl.BlockSpec((NB, SD), lambda i: (i, 0)),
      out_shape=jax.ShapeDtypeStruct((N_NODES, SD), jnp.float32),
  )(v, st, gamma, beta)


def _final_body(h_ref, agg_ref, w1_ref, b1_ref, w2_ref, b2_ref, batch_ref,
                fc1w_ref, fc1b_ref, fc2w_ref, fc2b_ref, out_ref, acc_ref):
  i = pl.program_id(0)
  z = h_ref[...] + agg_ref[0] + agg_ref[1]
  u = jnp.maximum(
      jnp.dot(z, w1_ref[...], preferred_element_type=jnp.float32)
      + b1_ref[...], 0.0)
  v = (jnp.dot(u, w2_ref[...], preferred_element_type=jnp.float32)
       + b2_ref[...])
  hout = jnp.maximum(v, 0.0)
  bids = batch_ref[0, 0, :]
  rows = jax.lax.broadcasted_iota(jnp.int32, (N_GRAPHS, NB), 0)
  onehot = (rows == bids[None, :]).astype(jnp.float32)
  contrib = jnp.dot(onehot, hout, preferred_element_type=jnp.float32)

  @pl.when(i == 0)
  def _():
    acc_ref[...] = contrib

  @pl.when(i > 0)
  def _():
    acc_ref[...] = acc_ref[...] + contrib

  @pl.when(i == NBLK - 1)
  def _():
    g = jnp.maximum(
        jnp.dot(acc_ref[...], fc1w_ref[...],
                preferred_element_type=jnp.float32) + fc1b_ref[...], 0.0)
    out_ref[...] = (jnp.dot(g, fc2w_ref[...],
                            preferred_element_type=jnp.float32)
                    + fc2b_ref[...])


def _final(h, agg, w1, b1, w2, b2, batch3, fc1w, fc1b, fc2w, fc2b):
  return pl.pallas_call(
      _final_body,
      grid=(NBLK,),
      in_specs=[
          pl.BlockSpec((NB, SD), lambda i: (i, 0)),
          pl.BlockSpec((NC, NB, SD), lambda i: (0, i, 0)),
          pl.BlockSpec((SD, SD), lambda i: (0, 0)),
          pl.BlockSpec((1, SD), lambda i: (0, 0)),
          pl.BlockSpec((SD, SD), lambda i: (0, 0)),
          pl.BlockSpec((1, SD), lambda i: (0, 0)),
          pl.BlockSpec((1, 1, NB), lambda i: (i, 0, 0)),
          pl.BlockSpec((SD, SD), lambda i: (0, 0)),
          pl.BlockSpec((1, SD), lambda i: (0, 0)),
          pl.BlockSpec((SD, SD), lambda i: (0, 0)),
          pl.BlockSpec((1, SD), lambda i: (0, 0)),
      ],
      out_specs=pl.BlockSpec((N_GRAPHS, SD), lambda i: (0, 0)),
      out_shape=jax.ShapeDtypeStruct((N_GRAPHS, SD), jnp.float32),
      scratch_shapes=[pltpu.VMEM((N_GRAPHS, SD), jnp.float32)],
  )(h, agg, w1, b1, w2, b2, batch3, fc1w, fc1b, fc2w, fc2b)


def kernel(x, edge_index, batch, emb, nn_in_W1, nn_in_b1, nn_in_W2, nn_in_b2,
           bn_gamma, bn_beta, nn_out_W1, nn_out_b1, nn_out_W2, nn_out_b2,
           fc1_W, fc1_b, fc2_W, fc2_b):
  f32 = jnp.float32
  src = edge_index[0]
  dst = edge_index[1]
  pad_iota = jnp.arange(E_PAD - N_EDGES, dtype=jnp.int32)
  srcp = jnp.concatenate(
      [src, pad_iota % N_NODES]).reshape(NW, BLKS, EBLK)
  dstp = jnp.concatenate(
      [dst, DUMMY + pad_iota % (ACC_ROWS - N_NODES)]
  ).reshape(NW, BLKS, EBLK)
  xi = jnp.concatenate(
      [jnp.squeeze(x, -1),
       jnp.arange(X_PAD - N_NODES, dtype=jnp.int32) % VOCAB]
  ).reshape(NW, XB, EBLK)
  zero_rows = jnp.zeros((ROWS_PER_TILE, SD), f32)

  w1i = nn_in_W1.T
  w2i = nn_in_W2.T
  w1o = nn_out_W1.T
  w2o = nn_out_W2.T
  b1i = nn_in_b1.reshape(1, SD)
  b2i = nn_in_b2.reshape(1, SD)
  b1o = nn_out_b1.reshape(1, SD)
  b2o = nn_out_b2.reshape(1, SD)
  gam = bn_gamma.reshape(1, SD)
  bet = bn_beta.reshape(1, SD)
  fc1T = fc1_W.T
  fc1b2 = fc1_b.reshape(1, SD)
  fc2T = jnp.zeros((SD, SD), f32).at[:, :N_CLASSES].set(fc2_W.T)
  fc2b2 = jnp.zeros((1, SD), f32).at[0, :N_CLASSES].set(fc2_b)
  batch3 = batch.reshape(NBLK, 1, NB)

  h = _embed(emb, xi)
  for _ in range(1 + HL):
    agg = _agg(h, srcp, dstp, zero_rows)
    v, st = _mlp(h, agg, w1i, b1i, w2i, b2i)
    h = _norm(v, st, gam, bet)
  agg = _agg(h, srcp, dstp, zero_rows)
  out = _final(h, agg, w1o, b1o, w2o, b2o, batch3, fc1T, fc1b2, fc2T, fc2b2)
  return out[:, :N_CLASSES]
